# Initial kernel scaffold; baseline (speedup 1.0000x reference)
import functools

import jax
import jax.numpy as jnp
from jax import lax
from jax.experimental import pallas as pl
from jax.experimental.pallas import tpu as pltpu
from jax.experimental.pallas import tpu_sc as plsc

B = 8
H = 224
W = 224
HW = H * W
N_M = 4096
N_U = 1024
NMH = N_M // 2
NUH = N_U // 2
PER_COMBO = 2 * NMH + 2 * NUH
PER_IMG = 2 * PER_COMBO
LANES = 16
MB_IT = NMH // LANES
UB_IT = NUH // LANES


def _sc_loss_kernel(fields_hbm, rows_hbm, cols_hbm, out_hbm,
                    idx_r_v, idx_c_v, field_v, valb_v, vald_v,
                    pairb_v, paird_v, part_v, red_v, out_v,
                    matched_sp, partials_sp):
    c = lax.axis_index("c")
    s = lax.axis_index("s")
    f = s % 2
    lb = (s // 2) % 4
    h = s // 8
    b = c * 4 + lb

    is_pred = f == 0

    fbase = (f * B + b) * HW
    pltpu.sync_copy(fields_hbm.at[pl.ds(fbase, HW)], field_v)
    ibase = (f * B + b) * PER_IMG + h * PER_COMBO
    pltpu.sync_copy(rows_hbm.at[pl.ds(ibase, PER_COMBO)], idx_r_v)
    pltpu.sync_copy(cols_hbm.at[pl.ds(ibase, PER_COMBO)], idx_c_v)

    def activate(v):
        return jnp.where(is_pred, 1.0 / (1.0 + jnp.exp(-v)), v)

    def gather_at(off):
        r = idx_r_v[pl.ds(off, LANES)]
        col = idx_c_v[pl.ds(off, LANES)]
        return plsc.load_gather(field_v, [r * W + col])

    def matched_body(i, _):
        o = i * LANES
        valb_v[pl.ds(o, LANES)] = activate(gather_at(o))
        vald_v[pl.ds(o, LANES)] = activate(gather_at(NMH + o))
        return 0
    lax.fori_loop(0, MB_IT, matched_body, 0)

    def unmatched_body(i, acc):
        o = i * LANES
        va = activate(gather_at(2 * NMH + o))
        vb = activate(gather_at(2 * NMH + NUH + o))
        d = va - vb
        return acc + d * d
    acc = lax.fori_loop(0, UB_IT, unmatched_body,
                        jnp.zeros((LANES,), jnp.float32))
    part_v[...] = acc

    @pl.when(jnp.logical_not(is_pred))
    def _publish():
        pltpu.sync_copy(valb_v, matched_sp.at[lb, 0, pl.ds(h * NMH, NMH)])
        pltpu.sync_copy(vald_v, matched_sp.at[lb, 1, pl.ds(h * NMH, NMH)])

    plsc.subcore_barrier()

    @pl.when(is_pred)
    def _matched_loss():
        pltpu.sync_copy(matched_sp.at[lb, 0, pl.ds(h * NMH, NMH)], pairb_v)
        pltpu.sync_copy(matched_sp.at[lb, 1, pl.ds(h * NMH, NMH)], paird_v)

        def body(i, acc):
            o = i * LANES
            db = valb_v[pl.ds(o, LANES)] - pairb_v[pl.ds(o, LANES)]
            dd = vald_v[pl.ds(o, LANES)] - paird_v[pl.ds(o, LANES)]
            return acc + 2.0 * (db * db + dd * dd)
        m_acc = lax.fori_loop(0, MB_IT, body,
                              jnp.zeros((LANES,), jnp.float32))
        part_v[...] = part_v[...] + m_acc

    pltpu.sync_copy(part_v, partials_sp.at[pl.ds(s * LANES, LANES)])
    plsc.subcore_barrier()

    @pl.when(s == 0)
    def _reduce():
        pltpu.sync_copy(partials_sp, red_v)

        def body(j, acc):
            return acc + red_v[pl.ds(j * LANES, LANES)]
        tot = lax.fori_loop(0, 16, body, jnp.zeros((LANES,), jnp.float32))
        total = jnp.sum(tot) * jnp.float32(1.0 / B)
        out_v[...] = jnp.broadcast_to(total, (LANES,))
        pltpu.sync_copy(out_v, out_hbm.at[c])


_sc_loss = functools.partial(
    pl.kernel,
    mesh=plsc.VectorSubcoreMesh(core_axis_name="c", subcore_axis_name="s"),
    out_type=jax.ShapeDtypeStruct((2, LANES), jnp.float32),
    scratch_types=[
        pltpu.VMEM((PER_COMBO,), jnp.int32),
        pltpu.VMEM((PER_COMBO,), jnp.int32),
        pltpu.VMEM((HW,), jnp.float32),
        pltpu.VMEM((NMH,), jnp.float32),
        pltpu.VMEM((NMH,), jnp.float32),
        pltpu.VMEM((NMH,), jnp.float32),
        pltpu.VMEM((NMH,), jnp.float32),
        pltpu.VMEM((LANES,), jnp.float32),
        pltpu.VMEM((16 * LANES,), jnp.float32),
        pltpu.VMEM((LANES,), jnp.float32),
        pltpu.VMEM_SHARED((4, 2, N_M), jnp.float32),
        pltpu.VMEM_SHARED((16 * LANES,), jnp.float32),
    ],
)(_sc_loss_kernel)


def _pack_indices(mb, md, ub, ud):
    parts = []
    for h in (0, 1):
        parts += [mb[:, h * NMH:(h + 1) * NMH], md[:, h * NMH:(h + 1) * NMH],
                  ub[:, h * NUH:(h + 1) * NUH], ud[:, h * NUH:(h + 1) * NUH]]
    return jnp.concatenate(parts, axis=1)


@jax.jit
def kernel(input, target, pred_mb, pred_md, tgt_mb, tgt_md,
           pred_ub, pred_ud, tgt_ub, tgt_ud):
    fields = jnp.concatenate(
        [input.reshape(B, HW), target.reshape(B, HW)], axis=0).reshape(-1)

    def comp(x, k):
        return x[..., k].astype(jnp.int32)

    pad = jnp.zeros((B, N_U - tgt_ub.shape[1]), jnp.int32)
    rows = jnp.concatenate([
        _pack_indices(comp(pred_mb, 0), comp(pred_md, 0),
                      comp(pred_ub, 0), comp(pred_ud, 0)),
        _pack_indices(comp(tgt_mb, 0), comp(tgt_md, 0),
                      jnp.concatenate([comp(tgt_ub, 0), pad], axis=1),
                      jnp.concatenate([comp(tgt_ud, 0), pad], axis=1)),
    ], axis=0).reshape(-1)
    cols = jnp.concatenate([
        _pack_indices(comp(pred_mb, 1), comp(pred_md, 1),
                      comp(pred_ub, 1), comp(pred_ud, 1)),
        _pack_indices(comp(tgt_mb, 1), comp(tgt_md, 1),
                      jnp.concatenate([comp(tgt_ub, 1), pad], axis=1),
                      jnp.concatenate([comp(tgt_ud, 1), pad], axis=1)),
    ], axis=0).reshape(-1)

    out = _sc_loss(fields, rows, cols)
    return out[0, 0] + out[1, 0]


# speedup vs baseline: 1.9015x; 1.9015x over previous
import functools

import jax
import jax.numpy as jnp
from jax import lax
from jax.experimental import pallas as pl
from jax.experimental.pallas import tpu as pltpu
from jax.experimental.pallas import tpu_sc as plsc

B = 8
H = 224
W = 224
HW = H * W
N_M = 4096
N_U = 1024
NMH = N_M // 2
NUH = N_U // 2
PER_COMBO = 2 * NMH + 2 * NUH
PER_IMG = 2 * PER_COMBO
LANES = 16
MB_IT = NMH // LANES
UB_IT = NUH // LANES


def _sc_loss_kernel(fields_hbm, rows_hbm, cols_hbm, out_hbm,
                    idx_r_v, idx_c_v, field_v, valb_v, vald_v,
                    pairb_v, paird_v, part_v, red_v, out_v,
                    matched_sp, partials_sp):
    c = lax.axis_index("c")
    s = lax.axis_index("s")
    f = s % 2
    lb = (s // 2) % 4
    h = s // 8
    b = c * 4 + lb

    is_pred = f == 0

    fbase = (f * B + b) * HW
    pltpu.sync_copy(fields_hbm.at[pl.ds(fbase, HW)], field_v)
    ibase = (f * B + b) * PER_IMG + h * PER_COMBO
    pltpu.sync_copy(rows_hbm.at[pl.ds(ibase, PER_COMBO)], idx_r_v)
    pltpu.sync_copy(cols_hbm.at[pl.ds(ibase, PER_COMBO)], idx_c_v)

    def activate(v):
        return jnp.where(is_pred, 1.0 / (1.0 + jnp.exp(-v)), v)

    def gather_at(off):
        r = idx_r_v[pl.ds(off, LANES)]
        col = idx_c_v[pl.ds(off, LANES)]
        return plsc.load_gather(field_v, [r * W + col])

    def matched_body(i, _):
        o = i * LANES
        valb_v[pl.ds(o, LANES)] = activate(gather_at(o))
        vald_v[pl.ds(o, LANES)] = activate(gather_at(NMH + o))
        return 0
    lax.fori_loop(0, MB_IT, matched_body, 0)

    def unmatched_body(i, acc):
        o = i * LANES
        va = activate(gather_at(2 * NMH + o))
        vb = activate(gather_at(2 * NMH + NUH + o))
        d = va - vb
        return acc + d * d
    acc = lax.fori_loop(0, UB_IT, unmatched_body,
                        jnp.zeros((LANES,), jnp.float32))
    part_v[...] = acc

    @pl.when(jnp.logical_not(is_pred))
    def _publish():
        pltpu.sync_copy(valb_v, matched_sp.at[lb, 0, pl.ds(h * NMH, NMH)])
        pltpu.sync_copy(vald_v, matched_sp.at[lb, 1, pl.ds(h * NMH, NMH)])

    plsc.subcore_barrier()

    @pl.when(is_pred)
    def _matched_loss():
        pltpu.sync_copy(matched_sp.at[lb, 0, pl.ds(h * NMH, NMH)], pairb_v)
        pltpu.sync_copy(matched_sp.at[lb, 1, pl.ds(h * NMH, NMH)], paird_v)

        def body(i, acc):
            o = i * LANES
            db = valb_v[pl.ds(o, LANES)] - pairb_v[pl.ds(o, LANES)]
            dd = vald_v[pl.ds(o, LANES)] - paird_v[pl.ds(o, LANES)]
            return acc + 2.0 * (db * db + dd * dd)
        m_acc = lax.fori_loop(0, MB_IT, body,
                              jnp.zeros((LANES,), jnp.float32))
        part_v[...] = part_v[...] + m_acc

    pltpu.sync_copy(part_v, partials_sp.at[pl.ds(s * LANES, LANES)])
    plsc.subcore_barrier()

    @pl.when(s == 0)
    def _reduce():
        pltpu.sync_copy(partials_sp, red_v)

        def body(j, acc):
            return acc + red_v[pl.ds(j * LANES, LANES)]
        tot = lax.fori_loop(0, 16, body, jnp.zeros((LANES,), jnp.float32))
        total = jnp.sum(tot) * jnp.float32(1.0 / B)
        out_v[...] = jnp.broadcast_to(total, (LANES,))
        pltpu.sync_copy(out_v, out_hbm.at[c])


_sc_loss = functools.partial(
    pl.kernel,
    mesh=plsc.VectorSubcoreMesh(core_axis_name="c", subcore_axis_name="s"),
    out_type=jax.ShapeDtypeStruct((2, LANES), jnp.float32),
    compiler_params=pltpu.CompilerParams(needs_layout_passes=False),
    scratch_types=[
        pltpu.VMEM((PER_COMBO,), jnp.int32),
        pltpu.VMEM((PER_COMBO,), jnp.int32),
        pltpu.VMEM((HW,), jnp.float32),
        pltpu.VMEM((NMH,), jnp.float32),
        pltpu.VMEM((NMH,), jnp.float32),
        pltpu.VMEM((NMH,), jnp.float32),
        pltpu.VMEM((NMH,), jnp.float32),
        pltpu.VMEM((LANES,), jnp.float32),
        pltpu.VMEM((16 * LANES,), jnp.float32),
        pltpu.VMEM((LANES,), jnp.float32),
        pltpu.VMEM_SHARED((4, 2, N_M), jnp.float32),
        pltpu.VMEM_SHARED((16 * LANES,), jnp.float32),
    ],
)(_sc_loss_kernel)


def _pack_indices(mb, md, ub, ud):
    parts = []
    for h in (0, 1):
        parts += [mb[:, h * NMH:(h + 1) * NMH], md[:, h * NMH:(h + 1) * NMH],
                  ub[:, h * NUH:(h + 1) * NUH], ud[:, h * NUH:(h + 1) * NUH]]
    return jnp.concatenate(parts, axis=1)


@jax.jit
def kernel(input, target, pred_mb, pred_md, tgt_mb, tgt_md,
           pred_ub, pred_ud, tgt_ub, tgt_ud):
    fields = jnp.concatenate(
        [input.reshape(B, HW), target.reshape(B, HW)], axis=0).reshape(-1)

    def comp(x, k):
        return x[..., k].astype(jnp.int32)

    pad = jnp.zeros((B, N_U - tgt_ub.shape[1]), jnp.int32)
    rows = jnp.concatenate([
        _pack_indices(comp(pred_mb, 0), comp(pred_md, 0),
                      comp(pred_ub, 0), comp(pred_ud, 0)),
        _pack_indices(comp(tgt_mb, 0), comp(tgt_md, 0),
                      jnp.concatenate([comp(tgt_ub, 0), pad], axis=1),
                      jnp.concatenate([comp(tgt_ud, 0), pad], axis=1)),
    ], axis=0).reshape(-1)
    cols = jnp.concatenate([
        _pack_indices(comp(pred_mb, 1), comp(pred_md, 1),
                      comp(pred_ub, 1), comp(pred_ud, 1)),
        _pack_indices(comp(tgt_mb, 1), comp(tgt_md, 1),
                      jnp.concatenate([comp(tgt_ub, 1), pad], axis=1),
                      jnp.concatenate([comp(tgt_ud, 1), pad], axis=1)),
    ], axis=0).reshape(-1)

    out = _sc_loss(fields, rows, cols)
    return out[0, 0] + out[1, 0]
